# baseline (speedup 1.0000x reference)
import functools
import math

import jax
import jax.numpy as jnp
from jax import lax
from jax.experimental import pallas as pl
from jax.experimental.pallas import tpu as pltpu
from jax.experimental.pallas import tpu_sc as plsc

_N = 100000
_E = 1600000
_D_IN = 37
_DH = 64
_EPS = 1e-05
_SW = 16
_NSLICE = _DH // _SW
_K = 400
_NSUB = 16
_EPT = _E // _NSUB
_NB = _EPT // _K
_RPT = 6256
_RPT_LAST = _N - 15 * _RPT
_NPAD = _RPT * _NSUB
_INV = 1.0 / math.sqrt(1.0 + _EPS)


def _make_segsum():
    mesh = plsc.VectorSubcoreMesh(core_axis_name="c", subcore_axis_name="s")
    out_type = [jax.ShapeDtypeStruct((_NPAD, _SW), jnp.float32)
                for _ in range(_NSLICE)]

    @functools.partial(
        pl.kernel, mesh=mesh, out_type=out_type,
        scratch_types=[
            pltpu.VMEM((_K,), jnp.int32),
            pltpu.VMEM((_K,), jnp.int32),
            pltpu.VMEM((_K,), jnp.int32),
            pltpu.VMEM((_K,), jnp.int32),
            pltpu.VMEM((_K, _SW), jnp.float32),
            pltpu.VMEM((_K, _SW), jnp.float32),
            pltpu.VMEM_SHARED((_NPAD, _SW), jnp.float32),
            pltpu.SemaphoreType.DMA,
            pltpu.SemaphoreType.DMA,
        ],
        compiler_params=pltpu.CompilerParams(use_tc_tiling_on_sc=False),
    )
    def segsum(f0, f1, f2, f3, src, dst, zeros, o0, o1, o2, o3,
               srcv0, dstv0, srcv1, dstv1, rows0, rows1, acc, sem0, sem1):
        c = lax.axis_index("c")
        s = lax.axis_index("s")
        ebase = s * _EPT
        r0 = s * _RPT
        feats = [f0, f1, f2, f3]
        outs = [o0, o1, o2, o3]

        def run_slice(feat, o):
            pltpu.sync_copy(zeros.at[pl.ds(r0, _RPT)], acc.at[pl.ds(r0, _RPT)])
            plsc.subcore_barrier()

            def body(j, carry):
                b0 = ebase + (2 * j) * _K
                b1 = b0 + _K
                pltpu.sync_copy(src.at[pl.ds(b0, _K)], srcv0)
                pltpu.sync_copy(dst.at[pl.ds(b0, _K)], dstv0)
                h0 = pltpu.async_copy(feat.at[srcv0], rows0, sem0)
                pltpu.sync_copy(src.at[pl.ds(b1, _K)], srcv1)
                pltpu.sync_copy(dst.at[pl.ds(b1, _K)], dstv1)
                h1 = pltpu.async_copy(feat.at[srcv1], rows1, sem1)
                h0.wait()
                pltpu.sync_copy(rows0, acc.at[dstv0], add=True)
                h1.wait()
                pltpu.sync_copy(rows1, acc.at[dstv1], add=True)
                return carry

            lax.fori_loop(0, _NB // 2, body, 0)
            if _NB % 2:
                bl_ = ebase + (_NB - 1) * _K
                pltpu.sync_copy(src.at[pl.ds(bl_, _K)], srcv0)
                pltpu.sync_copy(dst.at[pl.ds(bl_, _K)], dstv0)
                pltpu.async_copy(feat.at[srcv0], rows0, sem0).wait()
                pltpu.sync_copy(rows0, acc.at[dstv0], add=True)
            plsc.subcore_barrier()
            pltpu.sync_copy(acc.at[pl.ds(r0, _RPT)], o.at[pl.ds(r0, _RPT)])

        for ci in range(2):
            @pl.when(c == ci)
            def _():
                run_slice(feats[2 * ci], outs[2 * ci])
                run_slice(feats[2 * ci + 1], outs[2 * ci + 1])

    return segsum


_SEGSUM = _make_segsum()

_R = 2176
_GRID = _NPAD // _R


def _tc_layer(residual, full_out):
    def body(a0, a1, a2, a3, x0, x1, x2, x3, c3, wlt, bl, wrt, g, be, *outs):
        cnt = jnp.maximum(c3[...][:, _SW - 1:_SW], 1.0)
        mean = jnp.concatenate(
            [a0[...], a1[...], a2[...], a3[...]], axis=1) / cnt
        xp = jnp.concatenate([x0[...], x1[...], x2[...], x3[...]], axis=1)
        h = (jnp.dot(mean, wlt[...], preferred_element_type=jnp.float32)
             + bl[...]
             + jnp.dot(xp, wrt[...], preferred_element_type=jnp.float32))
        h = h * (g[...] * _INV) + be[...]
        h = jnp.maximum(h, 0.0)
        if residual:
            h = xp + 0.5 * h
        if full_out:
            outs[0][...] = h
        else:
            for j in range(_NSLICE):
                outs[j][...] = h[:, j * _SW:(j + 1) * _SW]

    slice_spec = pl.BlockSpec((_R, _SW), lambda i: (i, 0))
    w_spec = pl.BlockSpec((_DH, _DH), lambda i: (0, 0))
    v_spec = pl.BlockSpec((1, _DH), lambda i: (0, 0))
    in_specs = ([slice_spec] * 9 + [w_spec, v_spec, w_spec, v_spec, v_spec])
    if full_out:
        out_specs = pl.BlockSpec((_R, _DH), lambda i: (i, 0))
        out_shape = jax.ShapeDtypeStruct((_NPAD, _DH), jnp.float32)
    else:
        out_specs = [slice_spec] * _NSLICE
        out_shape = [jax.ShapeDtypeStruct((_NPAD, _SW), jnp.float32)
                     for _ in range(_NSLICE)]
    return pl.pallas_call(
        body, grid=(_GRID,), in_specs=in_specs,
        out_specs=out_specs, out_shape=out_shape)


_TC0 = _tc_layer(residual=False, full_out=False)
_TC1 = _tc_layer(residual=True, full_out=False)
_TC2 = _tc_layer(residual=False, full_out=True)


def _pad_w(w):
    return jnp.pad(w, ((0, 0), (0, _DH - w.shape[1])))


def kernel(x, edge_index, Wl0, bl0, Wr0, Wl1, bl1, Wr1, Wl2, bl2, Wr2,
           g0, be0, g1, be1, g2, be2):
    src = edge_index[0]
    dst = edge_index[1]
    xp = jnp.pad(x, ((0, _NPAD - _N), (0, _DH - _D_IN)))
    xp = xp.at[:, _DH - 1].set(1.0)
    zeros = jnp.zeros((_NPAD, _SW), jnp.float32)

    wl0t = _pad_w(Wl0).T
    wr0t = _pad_w(Wr0).T
    xs = [xp[:, j * _SW:(j + 1) * _SW] for j in range(_NSLICE)]

    agg0 = _SEGSUM(*xs, src, dst, zeros)
    c3 = agg0[3]
    h0 = _TC0(*agg0, *xs, c3, wl0t, bl0[None], wr0t, g0[None], be0[None])

    agg1 = _SEGSUM(*h0, src, dst, zeros)
    x2 = _TC1(*agg1, *h0, c3, Wl1.T, bl1[None], Wr1.T, g1[None], be1[None])

    agg2 = _SEGSUM(*x2, src, dst, zeros)
    out = _TC2(*agg2, *x2, c3, Wl2.T, bl2[None], Wr2.T, g2[None], be2[None])
    return out[:_N]


# speedup vs baseline: 1.2629x; 1.2629x over previous
import functools
import math

import jax
import jax.numpy as jnp
from jax import lax
from jax.experimental import pallas as pl
from jax.experimental.pallas import tpu as pltpu
from jax.experimental.pallas import tpu_sc as plsc

_N = 100000
_E = 1600000
_D_IN = 37
_DH = 64
_EPS = 1e-05
_SW = 16
_NSLICE = _DH // _SW
_K = 800
_NSUB = 16
_EPT = _E // _NSUB
_NB = _EPT // _K
_RPT = 6256
_RPT_LAST = _N - 15 * _RPT
_NPAD = _RPT * _NSUB
_INV = 1.0 / math.sqrt(1.0 + _EPS)


def _make_segsum():
    mesh = plsc.VectorSubcoreMesh(core_axis_name="c", subcore_axis_name="s")
    out_type = [jax.ShapeDtypeStruct((_NPAD, _SW), jnp.float32)
                for _ in range(_NSLICE)]

    @functools.partial(
        pl.kernel, mesh=mesh, out_type=out_type,
        scratch_types=[
            pltpu.VMEM((_K,), jnp.int32),
            pltpu.VMEM((_K,), jnp.int32),
            pltpu.VMEM((_K,), jnp.int32),
            pltpu.VMEM((_K,), jnp.int32),
            pltpu.VMEM((_K, _SW), jnp.float32),
            pltpu.VMEM((_K, _SW), jnp.float32),
            pltpu.VMEM_SHARED((_NPAD, _SW), jnp.float32),
            pltpu.SemaphoreType.DMA,
            pltpu.SemaphoreType.DMA,
        ],
        compiler_params=pltpu.CompilerParams(use_tc_tiling_on_sc=False),
    )
    def segsum(f0, f1, f2, f3, src, dst, zeros, o0, o1, o2, o3,
               srcv0, dstv0, srcv1, dstv1, rows0, rows1, acc, sem0, sem1):
        c = lax.axis_index("c")
        s = lax.axis_index("s")
        ebase = s * _EPT
        r0 = s * _RPT
        feats = [f0, f1, f2, f3]
        outs = [o0, o1, o2, o3]

        def run_slice(feat, o):
            pltpu.sync_copy(zeros.at[pl.ds(r0, _RPT)], acc.at[pl.ds(r0, _RPT)])
            plsc.subcore_barrier()

            def body(j, carry):
                b0 = ebase + (2 * j) * _K
                b1 = b0 + _K
                pltpu.sync_copy(src.at[pl.ds(b0, _K)], srcv0)
                pltpu.sync_copy(dst.at[pl.ds(b0, _K)], dstv0)
                h0 = pltpu.async_copy(feat.at[srcv0], rows0, sem0)
                pltpu.sync_copy(src.at[pl.ds(b1, _K)], srcv1)
                pltpu.sync_copy(dst.at[pl.ds(b1, _K)], dstv1)
                h1 = pltpu.async_copy(feat.at[srcv1], rows1, sem1)
                h0.wait()
                pltpu.sync_copy(rows0, acc.at[dstv0], add=True)
                h1.wait()
                pltpu.sync_copy(rows1, acc.at[dstv1], add=True)
                return carry

            lax.fori_loop(0, _NB // 2, body, 0)
            if _NB % 2:
                bl_ = ebase + (_NB - 1) * _K
                pltpu.sync_copy(src.at[pl.ds(bl_, _K)], srcv0)
                pltpu.sync_copy(dst.at[pl.ds(bl_, _K)], dstv0)
                pltpu.async_copy(feat.at[srcv0], rows0, sem0).wait()
                pltpu.sync_copy(rows0, acc.at[dstv0], add=True)
            plsc.subcore_barrier()
            pltpu.sync_copy(acc.at[pl.ds(r0, _RPT)], o.at[pl.ds(r0, _RPT)])

        for ci in range(2):
            @pl.when(c == ci)
            def _():
                run_slice(feats[2 * ci], outs[2 * ci])
                run_slice(feats[2 * ci + 1], outs[2 * ci + 1])

    return segsum


_SEGSUM = _make_segsum()

_R = 2176
_GRID = _NPAD // _R


def _tc_layer(residual, full_out):
    def body(a0, a1, a2, a3, x0, x1, x2, x3, c3, wlt, bl, wrt, g, be, *outs):
        cnt = jnp.maximum(c3[...][:, _SW - 1:_SW], 1.0)
        mean = jnp.concatenate(
            [a0[...], a1[...], a2[...], a3[...]], axis=1) / cnt
        xp = jnp.concatenate([x0[...], x1[...], x2[...], x3[...]], axis=1)
        h = (jnp.dot(mean, wlt[...], preferred_element_type=jnp.float32)
             + bl[...]
             + jnp.dot(xp, wrt[...], preferred_element_type=jnp.float32))
        h = h * (g[...] * _INV) + be[...]
        h = jnp.maximum(h, 0.0)
        if residual:
            h = xp + 0.5 * h
        if full_out:
            outs[0][...] = h
        else:
            for j in range(_NSLICE):
                outs[j][...] = h[:, j * _SW:(j + 1) * _SW]

    slice_spec = pl.BlockSpec((_R, _SW), lambda i: (i, 0))
    w_spec = pl.BlockSpec((_DH, _DH), lambda i: (0, 0))
    v_spec = pl.BlockSpec((1, _DH), lambda i: (0, 0))
    in_specs = ([slice_spec] * 9 + [w_spec, v_spec, w_spec, v_spec, v_spec])
    if full_out:
        out_specs = pl.BlockSpec((_R, _DH), lambda i: (i, 0))
        out_shape = jax.ShapeDtypeStruct((_NPAD, _DH), jnp.float32)
    else:
        out_specs = [slice_spec] * _NSLICE
        out_shape = [jax.ShapeDtypeStruct((_NPAD, _SW), jnp.float32)
                     for _ in range(_NSLICE)]
    return pl.pallas_call(
        body, grid=(_GRID,), in_specs=in_specs,
        out_specs=out_specs, out_shape=out_shape)


_TC0 = _tc_layer(residual=False, full_out=False)
_TC1 = _tc_layer(residual=True, full_out=False)
_TC2 = _tc_layer(residual=False, full_out=True)


def _pad_w(w):
    return jnp.pad(w, ((0, 0), (0, _DH - w.shape[1])))


def kernel(x, edge_index, Wl0, bl0, Wr0, Wl1, bl1, Wr1, Wl2, bl2, Wr2,
           g0, be0, g1, be1, g2, be2):
    src = edge_index[0]
    dst = edge_index[1]
    xp = jnp.pad(x, ((0, _NPAD - _N), (0, _DH - _D_IN)))
    xp = xp.at[:, _DH - 1].set(1.0)
    zeros = jnp.zeros((_NPAD, _SW), jnp.float32)

    wl0t = _pad_w(Wl0).T
    wr0t = _pad_w(Wr0).T
    xs = [xp[:, j * _SW:(j + 1) * _SW] for j in range(_NSLICE)]

    agg0 = _SEGSUM(*xs, src, dst, zeros)
    c3 = agg0[3]
    h0 = _TC0(*agg0, *xs, c3, wl0t, bl0[None], wr0t, g0[None], be0[None])

    agg1 = _SEGSUM(*h0, src, dst, zeros)
    x2 = _TC1(*agg1, *h0, c3, Wl1.T, bl1[None], Wr1.T, g1[None], be1[None])

    agg2 = _SEGSUM(*x2, src, dst, zeros)
    out = _TC2(*agg2, *x2, c3, Wl2.T, bl2[None], Wr2.T, g2[None], be2[None])
    return out[:_N]
